# baseline (speedup 1.0000x reference)
import functools

import jax
import jax.numpy as jnp
from jax import lax
from jax.experimental import pallas as pl
from jax.experimental.pallas import tpu as pltpu

M_BLK = 256
F_BLK = 256
_E = 8
_K = 2


def _moe_mlp_kernel(meta_ref, xs_ref, w1_ref, w1s_ref, w3_ref, w3s_ref,
                    w2_ref, w2s_ref, out_ref):
    f = pl.program_id(0)
    m = pl.program_id(1)
    n_active = meta_ref[meta_ref.shape[0] - 1]

    @pl.when(m < n_active)
    def _():
        x = xs_ref[...]
        w1 = w1_ref[0] * w1s_ref[0]
        w3 = w3_ref[0] * w3s_ref[0]
        w2 = w2_ref[0] * w2s_ref[0]
        dn = (((1,), (1,)), ((), ()))
        h1 = lax.dot_general(x, w1, dn, preferred_element_type=jnp.float32)
        h3 = lax.dot_general(x, w3, dn, preferred_element_type=jnp.float32)
        h = h1 * jax.nn.sigmoid(h1) * h3
        contrib = lax.dot_general(h, w2, dn,
                                  preferred_element_type=jnp.float32)
        sl = pl.ds(m * M_BLK, M_BLK)

        @pl.when(f == 0)
        def _():
            out_ref[sl, :] = contrib

        @pl.when(f != 0)
        def _():
            out_ref[sl, :] += contrib


@functools.partial(jax.jit, static_argnames=())
def kernel(hidden_states, gate_w, w1, w1_scale, w3, w3_scale, w2, w2_scale):
    b, s, d = hidden_states.shape
    e, f_dim, _ = w1.shape
    x = hidden_states.reshape(-1, d)
    t = x.shape[0]
    a = t * _K
    nb = (a + _E * (M_BLK - 1)) // M_BLK + 1
    p = nb * M_BLK
    nf = f_dim // F_BLK

    router_logits = x @ gate_w.T
    scores = jax.nn.sigmoid(router_logits)
    top_vals, top_idx = lax.top_k(scores, _K)
    routing_w = top_vals / jnp.sum(top_vals, axis=-1, keepdims=True)

    e_flat = top_idx.reshape(-1).astype(jnp.int32)
    oh = (e_flat[:, None] == jnp.arange(_E, dtype=jnp.int32)[None, :]
          ).astype(jnp.int32)
    csum = jnp.cumsum(oh, axis=0)
    counts = csum[-1]
    rank = jnp.take_along_axis(csum - oh, e_flat[:, None], axis=1)[:, 0]
    padded = ((counts + M_BLK - 1) // M_BLK) * M_BLK
    pad_cum = jnp.cumsum(padded)
    starts = pad_cum - padded
    dst = starts[e_flat] + rank
    tok_of_a = jnp.arange(a, dtype=jnp.int32) // _K
    src = jnp.zeros((p,), jnp.int32).at[dst].set(tok_of_a)
    block_expert = jnp.minimum(
        jnp.searchsorted(pad_cum, jnp.arange(nb, dtype=jnp.int32) * M_BLK,
                         side="right").astype(jnp.int32), _E - 1)
    n_active = (pad_cum[-1] // M_BLK).astype(jnp.int32)
    meta = jnp.concatenate([block_expert, n_active[None]])

    xs = x[src]

    rows = pl.pallas_call(
        _moe_mlp_kernel,
        grid_spec=pltpu.PrefetchScalarGridSpec(
            num_scalar_prefetch=1,
            grid=(nf, nb),
            in_specs=[
                pl.BlockSpec((M_BLK, d), lambda f, m, be: (m, 0)),
                pl.BlockSpec((1, F_BLK, d), lambda f, m, be: (be[m], f, 0)),
                pl.BlockSpec((1, F_BLK, d), lambda f, m, be: (be[m], f, 0)),
                pl.BlockSpec((1, F_BLK, d), lambda f, m, be: (be[m], f, 0)),
                pl.BlockSpec((1, F_BLK, d), lambda f, m, be: (be[m], f, 0)),
                pl.BlockSpec((1, d, F_BLK), lambda f, m, be: (be[m], 0, f)),
                pl.BlockSpec((1, d, F_BLK), lambda f, m, be: (be[m], 0, f)),
            ],
            out_specs=pl.BlockSpec((p, d), lambda f, m, be: (0, 0)),
        ),
        out_shape=jax.ShapeDtypeStruct((p, d), jnp.float32),
    )(meta, xs, w1, w1_scale, w3, w3_scale, w2, w2_scale)

    d0 = dst[0::2]
    d1 = dst[1::2]
    y = rows[d0] * routing_w[:, :1] + rows[d1] * routing_w[:, 1:]
    return y.reshape(b, s, d)
